# baseline (device time: 1068645 ns/iter reference)
import jax
import jax.numpy as jnp
import numpy as np
from jax import lax
from jax.experimental import pallas as pl
from jax.experimental.pallas import tpu as pltpu

N_DEV = 32
N_STREAMS = 6
N_SLOTS = 3

_sem_signal = getattr(pl, "semaphore_signal", None) or pltpu.semaphore_signal
_sem_wait = getattr(pl, "semaphore_wait", None) or pltpu.semaphore_wait
_DevIdType = getattr(pl, "DeviceIdType", None) or pltpu.DeviceIdType
_MESH = _DevIdType.MESH

_CYCLES = np.array([
    [0, 3, 4, 7, 15, 12, 11, 8, 16, 19, 20, 23, 31, 28, 27, 24,
     25, 26, 29, 30, 22, 21, 18, 17, 9, 10, 13, 14, 6, 5, 2, 1],
    [0, 8, 16, 24, 25, 17, 18, 26, 27, 28, 31, 30, 29, 21, 13, 14,
     22, 23, 15, 7, 6, 5, 4, 12, 20, 19, 11, 10, 9, 1, 2, 3],
    [0, 8, 11, 12, 20, 28, 29, 21, 22, 30, 31, 23, 15, 14, 6, 7,
     4, 3, 2, 5, 13, 10, 18, 19, 16, 24, 27, 26, 25, 17, 9, 1],
], dtype=np.int32)
_INVS = np.zeros_like(_CYCLES)
for _c in range(3):
    _INVS[_c, _CYCLES[_c]] = np.arange(N_DEV, dtype=np.int32)

_WIDTH_BLOCKS = [11, 11, 11, 10, 11, 10]
_BLK = 128
_WIDTHS = [w * _BLK for w in _WIDTH_BLOCKS]
_OFFS = np.concatenate([[0], np.cumsum(_WIDTHS)]).tolist()


def kernel(x, w_mat, scale_x, scale_w):
    m, k_loc = x.shape
    _, n = w_mat.shape
    chunk = m // N_DEV
    assert sum(_WIDTHS) == n

    my_log = lax.axis_index("i")
    id_list = []
    for c in range(3):
        rc = jnp.take(jnp.asarray(_INVS[c]), my_log)
        right = jnp.take(jnp.asarray(_CYCLES[c]), (rc + 1) % N_DEV)
        left = jnp.take(jnp.asarray(_CYCLES[c]), (rc + N_DEV - 1) % N_DEV)
        id_list += [rc, left, right]
    ids = jnp.stack(id_list).astype(jnp.int32)

    def body(x_ref, w_ref, sx_ref, sw_ref, ids_ref, out_ref, *scr):
        comm = scr[0:6]
        pbuf = scr[6:12]
        wb_ref = scr[12]
        send_sems = scr[13]
        recv_sems = scr[14]
        credit_sems = scr[15]
        store_sems = scr[16]

        scale = sx_ref[0] * sw_ref[0]

        def cyc_ids(c):
            return ids_ref[3 * c], ids_ref[3 * c + 1], ids_ref[3 * c + 2]

        def stream_peers(k):
            _, left, right = cyc_ids(k // 2)
            return (right, left) if k % 2 == 0 else (left, right)

        barrier_sem = pltpu.get_barrier_semaphore()
        for c in range(3):
            _, left, right = cyc_ids(c)
            _sem_signal(barrier_sem, inc=1, device_id=(left,),
                        device_id_type=_MESH)
            _sem_signal(barrier_sem, inc=1, device_id=(right,),
                        device_id_type=_MESH)
        _sem_wait(barrier_sem, 6)

        wb_ref[...] = w_ref[...].astype(jnp.bfloat16)

        def fill_partial(k, c_chunk):
            xc = x_ref[pl.ds(c_chunk * chunk, chunk), :].astype(jnp.bfloat16)
            wh = wb_ref[:, pl.ds(_OFFS[k], _WIDTHS[k])]
            pbuf[k][...] = jnp.dot(
                xc, wh, preferred_element_type=jnp.float32) * scale

        def rs_chunk(k, s):
            r = cyc_ids(k // 2)[0]
            if k % 2 == 0:
                return lax.rem(r - s - 1 + 2 * N_DEV, N_DEV)
            return lax.rem(r + s + 1, N_DEV)

        def ag_chunk(k, t):
            r = cyc_ids(k // 2)[0]
            if k % 2 == 0:
                return lax.rem(r - t + 2 * N_DEV, N_DEV)
            return lax.rem(r + t, N_DEV)

        def descriptor(k, g):
            down = stream_peers(k)[0]
            s0 = lax.rem(g + N_SLOTS - 1, N_SLOTS)
            s1 = lax.rem(g, N_SLOTS)
            return pltpu.make_async_remote_copy(
                src_ref=comm[k].at[s0],
                dst_ref=comm[k].at[s1],
                send_sem=send_sems.at[k, s0],
                recv_sem=recv_sems.at[k, s1],
                device_id=(down,),
                device_id_type=_MESH,
            )

        def store_desc(k, slot, c_store):
            return pltpu.make_async_copy(
                comm[k].at[slot],
                out_ref.at[pl.ds(c_store * chunk, chunk),
                           pl.ds(_OFFS[k], _WIDTHS[k])],
                store_sems.at[k],
            )

        def stream_step(k, g, is_rs, c_store):
            up = stream_peers(k)[1]
            recv_slot = lax.rem(g, N_SLOTS)
            d = descriptor(k, g)
            d.wait_recv()
            if is_rs:
                comm[k][recv_slot] = comm[k][recv_slot] + pbuf[k][...]
            else:
                st = store_desc(k, recv_slot, c_store)

                @pl.when(g >= N_DEV)
                def _():
                    st.wait()
                st.start()
            d.wait_send()
            _sem_signal(credit_sems.at[k], inc=1, device_id=(up,),
                        device_id_type=_MESH)

            @pl.when(g < 2 * (N_DEV - 1) - 1)
            def _():
                @pl.when(g >= N_SLOTS - 2)
                def _():
                    _sem_wait(credit_sems.at[k], 1)
                descriptor(k, g + 1).start()

        for k in range(N_STREAMS):
            fill_partial(k, cyc_ids(k // 2)[0])
            comm[k][N_SLOTS - 1] = pbuf[k][...]
        for k in range(N_STREAMS):
            descriptor(k, 0).start()

        def rs_step(s, carry):
            for k in range(N_STREAMS):
                fill_partial(k, rs_chunk(k, s))
            for k in range(N_STREAMS):
                stream_step(k, s, True, 0)
            return carry

        lax.fori_loop(0, N_DEV - 1, rs_step, 0)

        red_slot = (N_DEV - 2) % N_SLOTS
        for k in range(N_STREAMS):
            r = cyc_ids(k // 2)[0]
            red = lax.rem(r + 1, N_DEV) if k % 2 == 0 \
                else lax.rem(r + N_DEV - 1, N_DEV)
            st = store_desc(k, red_slot, red)
            st.start()
            st.wait()

        def ag_step(t, carry):
            g = t + N_DEV - 1
            for k in range(N_STREAMS):
                stream_step(k, g, False, ag_chunk(k, t))
            return carry

        lax.fori_loop(0, N_DEV - 1, ag_step, 0)

        for k in range(N_STREAMS):
            store_desc(k, 0, 0).wait()
            _sem_wait(credit_sems.at[k], N_SLOTS - 1)

    scratch = (
        [pltpu.VMEM((N_SLOTS, chunk, w), jnp.float32) for w in _WIDTHS]
        + [pltpu.VMEM((chunk, w), jnp.float32) for w in _WIDTHS]
        + [
            pltpu.VMEM((k_loc, n), jnp.bfloat16),
            pltpu.SemaphoreType.DMA((N_STREAMS, N_SLOTS)),
            pltpu.SemaphoreType.DMA((N_STREAMS, N_SLOTS)),
            pltpu.SemaphoreType.REGULAR((N_STREAMS,)),
            pltpu.SemaphoreType.DMA((N_STREAMS,)),
        ]
    )

    return pl.pallas_call(
        body,
        out_shape=jax.ShapeDtypeStruct((m, n), jnp.float32),
        in_specs=[
            pl.BlockSpec(memory_space=pltpu.VMEM),
            pl.BlockSpec(memory_space=pltpu.VMEM),
            pl.BlockSpec(memory_space=pltpu.SMEM),
            pl.BlockSpec(memory_space=pltpu.SMEM),
            pl.BlockSpec(memory_space=pltpu.SMEM),
        ],
        out_specs=pl.BlockSpec(memory_space=pl.ANY),
        scratch_shapes=scratch,
        compiler_params=pltpu.CompilerParams(collective_id=0),
    )(x, w_mat, scale_x, scale_w, ids)
